# baseline (speedup 1.0000x reference)
import functools

import jax
import jax.numpy as jnp
from jax import lax
from jax.experimental import pallas as pl
from jax.experimental.pallas import tpu as pltpu
from jax.experimental.pallas import tpu_sc as plsc

NC = 2
NS = 16
L = 16
CH = 128
RB = 512
C0_SHARE = 0.5


def _sc_mesh():
    return plsc.VectorSubcoreMesh(
        core_axis_name="c", subcore_axis_name="s", num_cores=NC, num_subcores=NS
    )


def _make_sc_deg(chunks, NP):
    rpt = NP // NS
    BK = 8
    cpt = chunks // (NC * NS)

    @functools.partial(
        pl.kernel,
        out_type=jax.ShapeDtypeStruct((NC, NP), jnp.float32),
        mesh=_sc_mesh(),
        scratch_types=[
            pltpu.VMEM_SHARED((NP,), jnp.float32),
            pltpu.VMEM((cpt, CH), jnp.int32),
            pltpu.VMEM((CH,), jnp.float32),
            pltpu.SemaphoreType.DMA,
        ],
    )
    def sc_deg(dst_ref, z1_ref, out_ref, dacc, didx, ones, sem):
        c = lax.axis_index("c")
        s = lax.axis_index("s")
        tid = c * NS + s
        r0 = s * rpt
        pltpu.sync_copy(z1_ref.at[pl.ds(r0, rpt)], dacc.at[pl.ds(r0, rpt)])
        pltpu.sync_copy(dst_ref.at[pl.ds(tid * cpt, cpt)], didx)
        for i in range(CH // L):
            ones[pl.ds(i * L, L)] = jnp.full((L,), 1.0, jnp.float32)
        plsc.subcore_barrier()

        def body(jb, carry):
            for i in range(BK):
                pltpu.async_copy(ones, dacc.at[didx.at[jb * BK + i]], sem, add=True)
            for i in range(BK):
                pltpu.make_async_copy(z1_ref.at[pl.ds(0, CH)], ones, sem).wait()
            return carry

        lax.fori_loop(0, cpt // BK, body, 0)
        plsc.subcore_barrier()
        pltpu.sync_copy(dacc.at[pl.ds(r0, rpt)], out_ref.at[c, pl.ds(r0, rpt)])

    return sc_deg


def _make_sc_mp(c0pt, c1pt, NP, D):
    rpt = NP // NS
    mxh = 32

    @functools.partial(
        pl.kernel,
        out_type=jax.ShapeDtypeStruct((NC, NP, D), jnp.float32),
        mesh=_sc_mesh(),
        scratch_types=[
            pltpu.VMEM_SHARED((NP, D), jnp.float32),
            pltpu.VMEM((mxh, CH), jnp.int32),
            pltpu.VMEM((mxh, CH), jnp.int32),
            pltpu.VMEM((CH, D), jnp.float32),
            pltpu.VMEM((CH, D), jnp.float32),
            pltpu.SemaphoreType.DMA,
            pltpu.SemaphoreType.DMA,
            pltpu.SemaphoreType.DMA,
            pltpu.SemaphoreType.DMA,
        ],
    )
    def sc_mp(src_ref, dst_ref, mp_ref, zero_ref, out_ref, acc,
              sidx, didx, rows0, rows1, sem0, sem1, ssem0, ssem1):
        c = lax.axis_index("c")
        s = lax.axis_index("s")
        r0 = s * rpt

        def run_edges(cpt, chunk_base):
            HF = mxh if cpt % mxh == 0 else 16
            for h in range(cpt // HF):
                pltpu.sync_copy(src_ref.at[pl.ds(chunk_base + h * HF, HF)],
                                sidx.at[pl.ds(0, HF)])
                pltpu.sync_copy(dst_ref.at[pl.ds(chunk_base + h * HF, HF)],
                                didx.at[pl.ds(0, HF)])
                if h == 0:
                    plsc.subcore_barrier()

                pltpu.async_copy(mp_ref.at[sidx.at[0]], rows0, sem0)
                pltpu.async_copy(mp_ref.at[sidx.at[1]], rows1, sem1)
                nj2 = HF // 2

                def body(j2, carry):
                    a = 2 * j2
                    pltpu.make_async_copy(zero_ref.at[pl.ds(0, CH)],
                                          rows0, sem0).wait()
                    pltpu.async_copy(rows0, acc.at[didx.at[a]], ssem0, add=True)
                    pltpu.make_async_copy(zero_ref.at[pl.ds(0, CH)],
                                          rows1, sem1).wait()
                    pltpu.async_copy(rows1, acc.at[didx.at[a + 1]], ssem1, add=True)

                    @pl.when(j2 < nj2 - 1)
                    def _():
                        pltpu.make_async_copy(zero_ref.at[pl.ds(0, CH)],
                                              rows0, ssem0).wait()
                        pltpu.async_copy(mp_ref.at[sidx.at[a + 2]], rows0, sem0)
                        pltpu.make_async_copy(zero_ref.at[pl.ds(0, CH)],
                                              rows1, ssem1).wait()
                        pltpu.async_copy(mp_ref.at[sidx.at[a + 3]], rows1, sem1)

                    return carry

                lax.fori_loop(0, nj2, body, 0)
                pltpu.make_async_copy(zero_ref.at[pl.ds(0, CH)], rows0, ssem0).wait()
                pltpu.make_async_copy(zero_ref.at[pl.ds(0, CH)], rows1, ssem1).wait()

        @pl.when(c == 0)
        def _():
            pltpu.sync_copy(mp_ref.at[pl.ds(r0, rpt)], acc.at[pl.ds(r0, rpt)])
            run_edges(c0pt, s * c0pt)

        @pl.when(c != 0)
        def _():
            pltpu.sync_copy(zero_ref.at[pl.ds(r0, rpt)], acc.at[pl.ds(r0, rpt)])
            run_edges(c1pt, NS * c0pt + s * c1pt)

        plsc.subcore_barrier()
        pltpu.sync_copy(acc.at[pl.ds(r0, rpt)], out_ref.at[c, pl.ds(r0, rpt)])

    return sc_mp


def _tc_first(xp, deg4, W, NP, D):
    G = NP // RB

    def body(x_ref, deg_ref, w_ref, o_ref):
        dinv = lax.rsqrt(deg_ref[0, 0, 0, :] + deg_ref[1, 0, 0, :] + 1.0)
        h = jnp.dot(x_ref[...], w_ref[...], preferred_element_type=jnp.float32)
        o_ref[...] = dinv[:, None] * h

    return pl.pallas_call(
        body,
        grid=(G,),
        in_specs=[
            pl.BlockSpec((RB, D), lambda i: (i, 0)),
            pl.BlockSpec((NC, 1, 1, RB), lambda i: (0, i, 0, 0)),
            pl.BlockSpec((D, D), lambda i: (0, 0)),
        ],
        out_specs=pl.BlockSpec((RB, D), lambda i: (i, 0)),
        out_shape=jax.ShapeDtypeStruct((NP, D), jnp.float32),
    )(xp, deg4, W)


def _tc_mid(accp, deg4, W, bias, NP, D):
    G = NP // RB

    def body(a_ref, deg_ref, w_ref, b_ref, o_ref):
        dinv = lax.rsqrt(deg_ref[0, 0, 0, :] + deg_ref[1, 0, 0, :] + 1.0)
        a = a_ref[0] + a_ref[1]
        z = jnp.maximum(dinv[:, None] * a + b_ref[...], 0.0)
        h = jnp.dot(z, w_ref[...], preferred_element_type=jnp.float32)
        o_ref[...] = dinv[:, None] * h

    return pl.pallas_call(
        body,
        grid=(G,),
        in_specs=[
            pl.BlockSpec((NC, RB, D), lambda i: (0, i, 0)),
            pl.BlockSpec((NC, 1, 1, RB), lambda i: (0, i, 0, 0)),
            pl.BlockSpec((D, D), lambda i: (0, 0)),
            pl.BlockSpec((1, D), lambda i: (0, 0)),
        ],
        out_specs=pl.BlockSpec((RB, D), lambda i: (i, 0)),
        out_shape=jax.ShapeDtypeStruct((NP, D), jnp.float32),
    )(accp, deg4, W, bias)


def _tc_last(accp, deg4, bias, NP, D):
    G = NP // RB

    def body(a_ref, deg_ref, b_ref, o_ref):
        dinv = lax.rsqrt(deg_ref[0, 0, 0, :] + deg_ref[1, 0, 0, :] + 1.0)
        a = a_ref[0] + a_ref[1]
        o_ref[...] = jnp.maximum(dinv[:, None] * a + b_ref[...], 0.0)

    return pl.pallas_call(
        body,
        grid=(G,),
        in_specs=[
            pl.BlockSpec((NC, RB, D), lambda i: (0, i, 0)),
            pl.BlockSpec((NC, 1, 1, RB), lambda i: (0, i, 0, 0)),
            pl.BlockSpec((1, D), lambda i: (0, 0)),
        ],
        out_specs=pl.BlockSpec((RB, D), lambda i: (i, 0)),
        out_shape=jax.ShapeDtypeStruct((NP, D), jnp.float32),
    )(accp, deg4, bias)


def kernel(x, edge_index, W1, b1, W2, b2, W3, b3):
    N, D = x.shape
    E = edge_index.shape[1]
    NP = -(-N // RB) * RB
    NT = NC * NS
    tot = -(-E // (NT * 16 * CH)) * 16 * NC
    c0pt = max(16, int(round(tot * C0_SHARE / 16.0)) * 16)
    c1pt = tot - c0pt
    chunks = (c0pt + c1pt) * NS
    ET = chunks * CH
    PAD = ET - E

    i32 = jnp.int32
    pad_iota = jnp.arange(PAD, dtype=i32)
    pad_rows = N + pad_iota % (NP - N)
    src = jnp.concatenate([edge_index[0].astype(i32), pad_rows])
    dst = jnp.concatenate([edge_index[1].astype(i32), pad_rows])
    src2 = src.reshape(chunks, CH)
    dst2 = dst.reshape(chunks, CH)
    xp = jnp.pad(x, ((0, NP - N), (0, 0)))
    z2 = jnp.zeros((NP, D), jnp.float32)
    z1 = jnp.zeros((NP,), jnp.float32)

    sc_deg = _make_sc_deg(chunks, NP)
    sc_mp = _make_sc_mp(c0pt, c1pt, NP, D)

    degp = sc_deg(dst2, z1)
    deg4 = degp.reshape(NC, NP // RB, 1, RB)

    m1 = _tc_first(xp, deg4, W1, NP, D)
    a1 = sc_mp(src2, dst2, m1, z2)
    m2 = _tc_mid(a1, deg4, W2, b1.reshape(1, D), NP, D)
    a2 = sc_mp(src2, dst2, m2, z2)
    m3 = _tc_mid(a2, deg4, W3, b2.reshape(1, D), NP, D)
    a3 = sc_mp(src2, dst2, m3, z2)
    out = _tc_last(a3, deg4, b3.reshape(1, D), NP, D)
    return out[:N][None, :, :]


# speedup vs baseline: 1.2108x; 1.2108x over previous
import functools

import jax
import jax.numpy as jnp
from jax import lax
from jax.experimental import pallas as pl
from jax.experimental.pallas import tpu as pltpu
from jax.experimental.pallas import tpu_sc as plsc

NC = 2
NS = 16
L = 16
CH = 128
RB = 512
C0_SHARE = 0.5


def _sc_mesh():
    return plsc.VectorSubcoreMesh(
        core_axis_name="c", subcore_axis_name="s", num_cores=NC, num_subcores=NS
    )


def _make_sc_deg(chunks, NP):
    rpt = NP // NS
    BK = 8
    cpt = chunks // (NC * NS)

    @functools.partial(
        pl.kernel,
        out_type=jax.ShapeDtypeStruct((NC, NP), jnp.float32),
        mesh=_sc_mesh(),
        scratch_types=[
            pltpu.VMEM_SHARED((NP,), jnp.float32),
            pltpu.VMEM((cpt, CH), jnp.int32),
            pltpu.VMEM((CH,), jnp.float32),
            pltpu.SemaphoreType.DMA,
        ],
    )
    def sc_deg(dst_ref, z1_ref, out_ref, dacc, didx, ones, sem):
        c = lax.axis_index("c")
        s = lax.axis_index("s")
        tid = c * NS + s
        r0 = s * rpt
        pltpu.sync_copy(z1_ref.at[pl.ds(r0, rpt)], dacc.at[pl.ds(r0, rpt)])
        pltpu.sync_copy(dst_ref.at[pl.ds(tid * cpt, cpt)], didx)
        for i in range(CH // L):
            ones[pl.ds(i * L, L)] = jnp.full((L,), 1.0, jnp.float32)
        plsc.subcore_barrier()

        def body(jb, carry):
            for i in range(BK):
                pltpu.async_copy(ones, dacc.at[didx.at[jb * BK + i]], sem, add=True)
            for i in range(BK):
                pltpu.make_async_copy(z1_ref.at[pl.ds(0, CH)], ones, sem).wait()
            return carry

        lax.fori_loop(0, cpt // BK, body, 0)
        plsc.subcore_barrier()
        pltpu.sync_copy(dacc.at[pl.ds(r0, rpt)], out_ref.at[c, pl.ds(r0, rpt)])

    return sc_deg


def _make_sc_mp(c0pt, c1pt, NP, D):
    rpt = NP // NS
    mxh = 32

    @functools.partial(
        pl.kernel,
        out_type=jax.ShapeDtypeStruct((NC, NP, D), jnp.float32),
        mesh=_sc_mesh(),
        scratch_types=[
            pltpu.VMEM_SHARED((NP, D), jnp.float32),
            pltpu.VMEM((mxh, CH), jnp.int32),
            pltpu.VMEM((mxh, CH), jnp.int32),
            pltpu.VMEM((CH, D), jnp.float32),
            pltpu.VMEM((CH, D), jnp.float32),
            pltpu.SemaphoreType.DMA,
            pltpu.SemaphoreType.DMA,
        ],
    )
    def sc_mp(src_ref, dst_ref, mp_ref, zero_ref, out_ref, acc,
              sidx, didx, rows0, rows1, sem0, sem1):
        c = lax.axis_index("c")
        s = lax.axis_index("s")
        r0 = s * rpt

        def run_edges(cpt, chunk_base):
            HF = mxh if cpt % mxh == 0 else 16
            for h in range(cpt // HF):
                pltpu.sync_copy(src_ref.at[pl.ds(chunk_base + h * HF, HF)],
                                sidx.at[pl.ds(0, HF)])
                pltpu.sync_copy(dst_ref.at[pl.ds(chunk_base + h * HF, HF)],
                                didx.at[pl.ds(0, HF)])
                if h == 0:
                    plsc.subcore_barrier()

                pltpu.async_copy(mp_ref.at[sidx.at[0]], rows0, sem0)
                pltpu.async_copy(mp_ref.at[sidx.at[1]], rows1, sem1)
                nj2 = HF // 2

                def body(j2, carry):
                    a = 2 * j2
                    pltpu.make_async_copy(zero_ref.at[pl.ds(0, CH)],
                                          rows0, sem0).wait()
                    pltpu.sync_copy(rows0, acc.at[didx.at[a]], add=True)

                    @pl.when(j2 < nj2 - 1)
                    def _():
                        pltpu.async_copy(mp_ref.at[sidx.at[a + 2]], rows0, sem0)

                    pltpu.make_async_copy(zero_ref.at[pl.ds(0, CH)],
                                          rows1, sem1).wait()
                    pltpu.sync_copy(rows1, acc.at[didx.at[a + 1]], add=True)

                    @pl.when(j2 < nj2 - 1)
                    def _():
                        pltpu.async_copy(mp_ref.at[sidx.at[a + 3]], rows1, sem1)

                    return carry

                lax.fori_loop(0, nj2, body, 0)

        @pl.when(c == 0)
        def _():
            pltpu.sync_copy(mp_ref.at[pl.ds(r0, rpt)], acc.at[pl.ds(r0, rpt)])
            run_edges(c0pt, s * c0pt)

        @pl.when(c != 0)
        def _():
            pltpu.sync_copy(zero_ref.at[pl.ds(r0, rpt)], acc.at[pl.ds(r0, rpt)])
            run_edges(c1pt, NS * c0pt + s * c1pt)

        plsc.subcore_barrier()
        pltpu.sync_copy(acc.at[pl.ds(r0, rpt)], out_ref.at[c, pl.ds(r0, rpt)])

    return sc_mp


def _tc_first(xp, deg4, W, NP, D):
    G = NP // RB

    def body(x_ref, deg_ref, w_ref, o_ref):
        dinv = lax.rsqrt(deg_ref[0, 0, 0, :] + deg_ref[1, 0, 0, :] + 1.0)
        h = jnp.dot(x_ref[...], w_ref[...], preferred_element_type=jnp.float32)
        o_ref[...] = dinv[:, None] * h

    return pl.pallas_call(
        body,
        grid=(G,),
        in_specs=[
            pl.BlockSpec((RB, D), lambda i: (i, 0)),
            pl.BlockSpec((NC, 1, 1, RB), lambda i: (0, i, 0, 0)),
            pl.BlockSpec((D, D), lambda i: (0, 0)),
        ],
        out_specs=pl.BlockSpec((RB, D), lambda i: (i, 0)),
        out_shape=jax.ShapeDtypeStruct((NP, D), jnp.float32),
    )(xp, deg4, W)


def _tc_mid(accp, deg4, W, bias, NP, D):
    G = NP // RB

    def body(a_ref, deg_ref, w_ref, b_ref, o_ref):
        dinv = lax.rsqrt(deg_ref[0, 0, 0, :] + deg_ref[1, 0, 0, :] + 1.0)
        a = a_ref[0] + a_ref[1]
        z = jnp.maximum(dinv[:, None] * a + b_ref[...], 0.0)
        h = jnp.dot(z, w_ref[...], preferred_element_type=jnp.float32)
        o_ref[...] = dinv[:, None] * h

    return pl.pallas_call(
        body,
        grid=(G,),
        in_specs=[
            pl.BlockSpec((NC, RB, D), lambda i: (0, i, 0)),
            pl.BlockSpec((NC, 1, 1, RB), lambda i: (0, i, 0, 0)),
            pl.BlockSpec((D, D), lambda i: (0, 0)),
            pl.BlockSpec((1, D), lambda i: (0, 0)),
        ],
        out_specs=pl.BlockSpec((RB, D), lambda i: (i, 0)),
        out_shape=jax.ShapeDtypeStruct((NP, D), jnp.float32),
    )(accp, deg4, W, bias)


def _tc_last(accp, deg4, bias, NP, D):
    G = NP // RB

    def body(a_ref, deg_ref, b_ref, o_ref):
        dinv = lax.rsqrt(deg_ref[0, 0, 0, :] + deg_ref[1, 0, 0, :] + 1.0)
        a = a_ref[0] + a_ref[1]
        o_ref[...] = jnp.maximum(dinv[:, None] * a + b_ref[...], 0.0)

    return pl.pallas_call(
        body,
        grid=(G,),
        in_specs=[
            pl.BlockSpec((NC, RB, D), lambda i: (0, i, 0)),
            pl.BlockSpec((NC, 1, 1, RB), lambda i: (0, i, 0, 0)),
            pl.BlockSpec((1, D), lambda i: (0, 0)),
        ],
        out_specs=pl.BlockSpec((RB, D), lambda i: (i, 0)),
        out_shape=jax.ShapeDtypeStruct((NP, D), jnp.float32),
    )(accp, deg4, bias)


def kernel(x, edge_index, W1, b1, W2, b2, W3, b3):
    N, D = x.shape
    E = edge_index.shape[1]
    NP = -(-N // RB) * RB
    NT = NC * NS
    tot = -(-E // (NT * 16 * CH)) * 16 * NC
    c0pt = max(16, int(round(tot * C0_SHARE / 16.0)) * 16)
    c1pt = tot - c0pt
    chunks = (c0pt + c1pt) * NS
    ET = chunks * CH
    PAD = ET - E

    i32 = jnp.int32
    pad_iota = jnp.arange(PAD, dtype=i32)
    pad_rows = N + pad_iota % (NP - N)
    src = jnp.concatenate([edge_index[0].astype(i32), pad_rows])
    dst = jnp.concatenate([edge_index[1].astype(i32), pad_rows])
    src2 = src.reshape(chunks, CH)
    dst2 = dst.reshape(chunks, CH)
    xp = jnp.pad(x, ((0, NP - N), (0, 0)))
    z2 = jnp.zeros((NP, D), jnp.float32)
    z1 = jnp.zeros((NP,), jnp.float32)

    sc_deg = _make_sc_deg(chunks, NP)
    sc_mp = _make_sc_mp(c0pt, c1pt, NP, D)

    degp = sc_deg(dst2, z1)
    deg4 = degp.reshape(NC, NP // RB, 1, RB)

    m1 = _tc_first(xp, deg4, W1, NP, D)
    a1 = sc_mp(src2, dst2, m1, z2)
    m2 = _tc_mid(a1, deg4, W2, b1.reshape(1, D), NP, D)
    a2 = sc_mp(src2, dst2, m2, z2)
    m3 = _tc_mid(a2, deg4, W3, b2.reshape(1, D), NP, D)
    a3 = sc_mp(src2, dst2, m3, z2)
    out = _tc_last(a3, deg4, b3.reshape(1, D), NP, D)
    return out[:N][None, :, :]


# speedup vs baseline: 1.2900x; 1.0654x over previous
import functools

import jax
import jax.numpy as jnp
from jax import lax
from jax.experimental import pallas as pl
from jax.experimental.pallas import tpu as pltpu
from jax.experimental.pallas import tpu_sc as plsc

NC = 2
NS = 16
L = 16
CH = 128
RB = 512
C0_SHARE = 0.5


def _sc_mesh():
    return plsc.VectorSubcoreMesh(
        core_axis_name="c", subcore_axis_name="s", num_cores=NC, num_subcores=NS
    )


def _make_sc_deg(chunks, NP):
    rpt = NP // NS
    BK = 8
    cpt = chunks // (NC * NS)

    @functools.partial(
        pl.kernel,
        out_type=jax.ShapeDtypeStruct((NC, NP), jnp.float32),
        mesh=_sc_mesh(),
        scratch_types=[
            pltpu.VMEM_SHARED((NP,), jnp.float32),
            pltpu.VMEM((cpt, CH), jnp.int32),
            pltpu.VMEM((CH,), jnp.float32),
            pltpu.SemaphoreType.DMA,
        ],
    )
    def sc_deg(dst_ref, z1_ref, out_ref, dacc, didx, ones, sem):
        c = lax.axis_index("c")
        s = lax.axis_index("s")
        tid = c * NS + s
        r0 = s * rpt
        pltpu.sync_copy(z1_ref.at[pl.ds(r0, rpt)], dacc.at[pl.ds(r0, rpt)])
        pltpu.sync_copy(dst_ref.at[pl.ds(tid * cpt, cpt)], didx)
        for i in range(CH // L):
            ones[pl.ds(i * L, L)] = jnp.full((L,), 1.0, jnp.float32)
        plsc.subcore_barrier()

        def body(jb, carry):
            for i in range(BK):
                pltpu.async_copy(ones, dacc.at[didx.at[jb * BK + i]], sem, add=True)
            for i in range(BK):
                pltpu.make_async_copy(z1_ref.at[pl.ds(0, CH)], ones, sem).wait()
            return carry

        lax.fori_loop(0, cpt // BK, body, 0)
        plsc.subcore_barrier()
        pltpu.sync_copy(dacc.at[pl.ds(r0, rpt)], out_ref.at[c, pl.ds(r0, rpt)])

    return sc_deg


def _make_sc_mp(c0pt, c1pt, NP, D):
    rpt = NP // NS
    mxh = 40

    @functools.partial(
        pl.kernel,
        out_type=jax.ShapeDtypeStruct((NC, NP, D), jnp.float32),
        mesh=_sc_mesh(),
        scratch_types=[
            pltpu.VMEM_SHARED((NP, D), jnp.float32),
            pltpu.VMEM((mxh, CH), jnp.int32),
            pltpu.VMEM((mxh, CH), jnp.int32),
            pltpu.VMEM((CH, D), jnp.float32),
            pltpu.VMEM((CH, D), jnp.float32),
            pltpu.SemaphoreType.DMA,
            pltpu.SemaphoreType.DMA,
        ],
    )
    def sc_mp(src_ref, dst_ref, mp_ref, zero_ref, out_ref, acc,
              sidx, didx, rows0, rows1, sem0, sem1):
        c = lax.axis_index("c")
        s = lax.axis_index("s")
        r0 = s * rpt

        def run_edges(cpt, chunk_base):
            HF = mxh if cpt % mxh == 0 else (32 if cpt % 32 == 0 else 16)
            for h in range(cpt // HF):
                pltpu.sync_copy(src_ref.at[pl.ds(chunk_base + h * HF, HF)],
                                sidx.at[pl.ds(0, HF)])
                pltpu.sync_copy(dst_ref.at[pl.ds(chunk_base + h * HF, HF)],
                                didx.at[pl.ds(0, HF)])
                if h == 0:
                    plsc.subcore_barrier()

                pltpu.async_copy(mp_ref.at[sidx.at[0]], rows0, sem0)
                pltpu.async_copy(mp_ref.at[sidx.at[1]], rows1, sem1)
                nj2 = HF // 2

                def body(j2, carry):
                    a = 2 * j2
                    pltpu.make_async_copy(zero_ref.at[pl.ds(0, CH)],
                                          rows0, sem0).wait()
                    pltpu.sync_copy(rows0, acc.at[didx.at[a]], add=True)

                    @pl.when(j2 < nj2 - 1)
                    def _():
                        pltpu.async_copy(mp_ref.at[sidx.at[a + 2]], rows0, sem0)

                    pltpu.make_async_copy(zero_ref.at[pl.ds(0, CH)],
                                          rows1, sem1).wait()
                    pltpu.sync_copy(rows1, acc.at[didx.at[a + 1]], add=True)

                    @pl.when(j2 < nj2 - 1)
                    def _():
                        pltpu.async_copy(mp_ref.at[sidx.at[a + 3]], rows1, sem1)

                    return carry

                lax.fori_loop(0, nj2, body, 0)

        @pl.when(c == 0)
        def _():
            pltpu.sync_copy(mp_ref.at[pl.ds(r0, rpt)], acc.at[pl.ds(r0, rpt)])
            run_edges(c0pt, s * c0pt)

        @pl.when(c != 0)
        def _():
            pltpu.sync_copy(zero_ref.at[pl.ds(r0, rpt)], acc.at[pl.ds(r0, rpt)])
            run_edges(c1pt, NS * c0pt + s * c1pt)

        plsc.subcore_barrier()
        pltpu.sync_copy(acc.at[pl.ds(r0, rpt)], out_ref.at[c, pl.ds(r0, rpt)])

    return sc_mp


def _tc_first(xp, deg4, W, NP, D):
    G = NP // RB

    def body(x_ref, deg_ref, w_ref, o_ref):
        dinv = lax.rsqrt(deg_ref[0, 0, 0, :] + deg_ref[1, 0, 0, :] + 1.0)
        h = jnp.dot(x_ref[...], w_ref[...], preferred_element_type=jnp.float32)
        o_ref[...] = dinv[:, None] * h

    return pl.pallas_call(
        body,
        grid=(G,),
        in_specs=[
            pl.BlockSpec((RB, D), lambda i: (i, 0)),
            pl.BlockSpec((NC, 1, 1, RB), lambda i: (0, i, 0, 0)),
            pl.BlockSpec((D, D), lambda i: (0, 0)),
        ],
        out_specs=pl.BlockSpec((RB, D), lambda i: (i, 0)),
        out_shape=jax.ShapeDtypeStruct((NP, D), jnp.float32),
    )(xp, deg4, W)


def _tc_mid(accp, deg4, W, bias, NP, D):
    G = NP // RB

    def body(a_ref, deg_ref, w_ref, b_ref, o_ref):
        dinv = lax.rsqrt(deg_ref[0, 0, 0, :] + deg_ref[1, 0, 0, :] + 1.0)
        a = a_ref[0] + a_ref[1]
        z = jnp.maximum(dinv[:, None] * a + b_ref[...], 0.0)
        h = jnp.dot(z, w_ref[...], preferred_element_type=jnp.float32)
        o_ref[...] = dinv[:, None] * h

    return pl.pallas_call(
        body,
        grid=(G,),
        in_specs=[
            pl.BlockSpec((NC, RB, D), lambda i: (0, i, 0)),
            pl.BlockSpec((NC, 1, 1, RB), lambda i: (0, i, 0, 0)),
            pl.BlockSpec((D, D), lambda i: (0, 0)),
            pl.BlockSpec((1, D), lambda i: (0, 0)),
        ],
        out_specs=pl.BlockSpec((RB, D), lambda i: (i, 0)),
        out_shape=jax.ShapeDtypeStruct((NP, D), jnp.float32),
    )(accp, deg4, W, bias)


def _tc_last(accp, deg4, bias, NP, D):
    G = NP // RB

    def body(a_ref, deg_ref, b_ref, o_ref):
        dinv = lax.rsqrt(deg_ref[0, 0, 0, :] + deg_ref[1, 0, 0, :] + 1.0)
        a = a_ref[0] + a_ref[1]
        o_ref[...] = jnp.maximum(dinv[:, None] * a + b_ref[...], 0.0)

    return pl.pallas_call(
        body,
        grid=(G,),
        in_specs=[
            pl.BlockSpec((NC, RB, D), lambda i: (0, i, 0)),
            pl.BlockSpec((NC, 1, 1, RB), lambda i: (0, i, 0, 0)),
            pl.BlockSpec((1, D), lambda i: (0, 0)),
        ],
        out_specs=pl.BlockSpec((RB, D), lambda i: (i, 0)),
        out_shape=jax.ShapeDtypeStruct((NP, D), jnp.float32),
    )(accp, deg4, bias)


def kernel(x, edge_index, W1, b1, W2, b2, W3, b3):
    N, D = x.shape
    E = edge_index.shape[1]
    NP = -(-N // RB) * RB
    NT = NC * NS
    tot = -(-E // (NT * 16 * CH)) * 16 * NC
    c0pt = max(16, int(round(tot * C0_SHARE / 16.0)) * 16)
    c1pt = tot - c0pt
    chunks = (c0pt + c1pt) * NS
    ET = chunks * CH
    PAD = ET - E

    i32 = jnp.int32
    pad_iota = jnp.arange(PAD, dtype=i32)
    pad_rows = N + pad_iota % (NP - N)
    src = jnp.concatenate([edge_index[0].astype(i32), pad_rows])
    dst = jnp.concatenate([edge_index[1].astype(i32), pad_rows])
    src2 = src.reshape(chunks, CH)
    dst2 = dst.reshape(chunks, CH)
    xp = jnp.pad(x, ((0, NP - N), (0, 0)))
    z2 = jnp.zeros((NP, D), jnp.float32)
    z1 = jnp.zeros((NP,), jnp.float32)

    sc_deg = _make_sc_deg(chunks, NP)
    sc_mp = _make_sc_mp(c0pt, c1pt, NP, D)

    degp = sc_deg(dst2, z1)
    deg4 = degp.reshape(NC, NP // RB, 1, RB)

    m1 = _tc_first(xp, deg4, W1, NP, D)
    a1 = sc_mp(src2, dst2, m1, z2)
    m2 = _tc_mid(a1, deg4, W2, b1.reshape(1, D), NP, D)
    a2 = sc_mp(src2, dst2, m2, z2)
    m3 = _tc_mid(a2, deg4, W3, b2.reshape(1, D), NP, D)
    a3 = sc_mp(src2, dst2, m3, z2)
    out = _tc_last(a3, deg4, b3.reshape(1, D), NP, D)
    return out[:N][None, :, :]


# speedup vs baseline: 1.3490x; 1.0457x over previous
import functools

import jax
import jax.numpy as jnp
from jax import lax
from jax.experimental import pallas as pl
from jax.experimental.pallas import tpu as pltpu
from jax.experimental.pallas import tpu_sc as plsc

NC = 2
NS = 16
L = 16
CH = 128
RB = 1024
C0_SHARE = 0.5


def _sc_mesh():
    return plsc.VectorSubcoreMesh(
        core_axis_name="c", subcore_axis_name="s", num_cores=NC, num_subcores=NS
    )


def _make_sc_deg(chunks, NP):
    rpt = NP // NS
    BK = 8
    cpt = chunks // (NC * NS)

    @functools.partial(
        pl.kernel,
        out_type=jax.ShapeDtypeStruct((NC, NP), jnp.float32),
        mesh=_sc_mesh(),
        scratch_types=[
            pltpu.VMEM_SHARED((NP,), jnp.float32),
            pltpu.VMEM((cpt, CH), jnp.int32),
            pltpu.VMEM((CH,), jnp.float32),
            pltpu.SemaphoreType.DMA,
        ],
    )
    def sc_deg(dst_ref, z1_ref, out_ref, dacc, didx, ones, sem):
        c = lax.axis_index("c")
        s = lax.axis_index("s")
        tid = c * NS + s
        r0 = s * rpt
        pltpu.sync_copy(z1_ref.at[pl.ds(r0, rpt)], dacc.at[pl.ds(r0, rpt)])
        pltpu.sync_copy(dst_ref.at[pl.ds(tid * cpt, cpt)], didx)
        for i in range(CH // L):
            ones[pl.ds(i * L, L)] = jnp.full((L,), 1.0, jnp.float32)
        plsc.subcore_barrier()

        def body(jb, carry):
            for i in range(BK):
                pltpu.async_copy(ones, dacc.at[didx.at[jb * BK + i]], sem, add=True)
            for i in range(BK):
                pltpu.make_async_copy(z1_ref.at[pl.ds(0, CH)], ones, sem).wait()
            return carry

        lax.fori_loop(0, cpt // BK, body, 0)
        plsc.subcore_barrier()
        pltpu.sync_copy(dacc.at[pl.ds(r0, rpt)], out_ref.at[c, pl.ds(r0, rpt)])

    return sc_deg


def _make_sc_mp(c0pt, c1pt, NP, D):
    rpt = NP // NS
    mxh = 40

    @functools.partial(
        pl.kernel,
        out_type=jax.ShapeDtypeStruct((NC, NP, D), jnp.float32),
        mesh=_sc_mesh(),
        scratch_types=[
            pltpu.VMEM_SHARED((NP, D), jnp.float32),
            pltpu.VMEM((mxh, CH), jnp.int32),
            pltpu.VMEM((mxh, CH), jnp.int32),
            pltpu.VMEM((CH, D), jnp.float32),
            pltpu.VMEM((CH, D), jnp.float32),
            pltpu.SemaphoreType.DMA,
            pltpu.SemaphoreType.DMA,
        ],
    )
    def sc_mp(src_ref, dst_ref, mp_ref, zero_ref, out_ref, acc,
              sidx, didx, rows0, rows1, sem0, sem1):
        c = lax.axis_index("c")
        s = lax.axis_index("s")
        r0 = s * rpt

        def run_edges(cpt, chunk_base):
            HF = mxh if cpt % mxh == 0 else (32 if cpt % 32 == 0 else 16)
            for h in range(cpt // HF):
                pltpu.sync_copy(src_ref.at[pl.ds(chunk_base + h * HF, HF)],
                                sidx.at[pl.ds(0, HF)])
                pltpu.sync_copy(dst_ref.at[pl.ds(chunk_base + h * HF, HF)],
                                didx.at[pl.ds(0, HF)])
                if h == 0:
                    plsc.subcore_barrier()

                pltpu.async_copy(mp_ref.at[sidx.at[0]], rows0, sem0)
                pltpu.async_copy(mp_ref.at[sidx.at[1]], rows1, sem1)
                nj2 = HF // 2

                def body(j2, carry):
                    a = 2 * j2
                    pltpu.make_async_copy(zero_ref.at[pl.ds(0, CH)],
                                          rows0, sem0).wait()
                    pltpu.sync_copy(rows0, acc.at[didx.at[a]], add=True)

                    @pl.when(j2 < nj2 - 1)
                    def _():
                        pltpu.async_copy(mp_ref.at[sidx.at[a + 2]], rows0, sem0)

                    pltpu.make_async_copy(zero_ref.at[pl.ds(0, CH)],
                                          rows1, sem1).wait()
                    pltpu.sync_copy(rows1, acc.at[didx.at[a + 1]], add=True)

                    @pl.when(j2 < nj2 - 1)
                    def _():
                        pltpu.async_copy(mp_ref.at[sidx.at[a + 3]], rows1, sem1)

                    return carry

                lax.fori_loop(0, nj2, body, 0)

        @pl.when(c == 0)
        def _():
            pltpu.sync_copy(mp_ref.at[pl.ds(r0, rpt)], acc.at[pl.ds(r0, rpt)])
            run_edges(c0pt, s * c0pt)

        @pl.when(c != 0)
        def _():
            pltpu.sync_copy(zero_ref.at[pl.ds(r0, rpt)], acc.at[pl.ds(r0, rpt)])
            run_edges(c1pt, NS * c0pt + s * c1pt)

        plsc.subcore_barrier()
        pltpu.sync_copy(acc.at[pl.ds(r0, rpt)], out_ref.at[c, pl.ds(r0, rpt)])

    return sc_mp


def _tc_first(xp, deg4, W, NP, D):
    G = NP // RB

    def body(x_ref, deg_ref, w_ref, o_ref):
        dinv = lax.rsqrt(deg_ref[0, 0, 0, :] + deg_ref[1, 0, 0, :] + 1.0)
        h = jnp.dot(x_ref[...], w_ref[...], preferred_element_type=jnp.float32)
        o_ref[...] = dinv[:, None] * h

    return pl.pallas_call(
        body,
        grid=(G,),
        in_specs=[
            pl.BlockSpec((RB, D), lambda i: (i, 0)),
            pl.BlockSpec((NC, 1, 1, RB), lambda i: (0, i, 0, 0)),
            pl.BlockSpec((D, D), lambda i: (0, 0)),
        ],
        out_specs=pl.BlockSpec((RB, D), lambda i: (i, 0)),
        out_shape=jax.ShapeDtypeStruct((NP, D), jnp.float32),
    )(xp, deg4, W)


def _tc_mid(accp, deg4, W, bias, NP, D):
    G = NP // RB

    def body(a_ref, deg_ref, w_ref, b_ref, o_ref):
        dinv = lax.rsqrt(deg_ref[0, 0, 0, :] + deg_ref[1, 0, 0, :] + 1.0)
        a = a_ref[0] + a_ref[1]
        z = jnp.maximum(dinv[:, None] * a + b_ref[...], 0.0)
        h = jnp.dot(z, w_ref[...], preferred_element_type=jnp.float32)
        o_ref[...] = dinv[:, None] * h

    return pl.pallas_call(
        body,
        grid=(G,),
        in_specs=[
            pl.BlockSpec((NC, RB, D), lambda i: (0, i, 0)),
            pl.BlockSpec((NC, 1, 1, RB), lambda i: (0, i, 0, 0)),
            pl.BlockSpec((D, D), lambda i: (0, 0)),
            pl.BlockSpec((1, D), lambda i: (0, 0)),
        ],
        out_specs=pl.BlockSpec((RB, D), lambda i: (i, 0)),
        out_shape=jax.ShapeDtypeStruct((NP, D), jnp.float32),
    )(accp, deg4, W, bias)


def _tc_last(accp, deg4, bias, NP, D):
    G = NP // RB

    def body(a_ref, deg_ref, b_ref, o_ref):
        dinv = lax.rsqrt(deg_ref[0, 0, 0, :] + deg_ref[1, 0, 0, :] + 1.0)
        a = a_ref[0] + a_ref[1]
        o_ref[...] = jnp.maximum(dinv[:, None] * a + b_ref[...], 0.0)

    return pl.pallas_call(
        body,
        grid=(G,),
        in_specs=[
            pl.BlockSpec((NC, RB, D), lambda i: (0, i, 0)),
            pl.BlockSpec((NC, 1, 1, RB), lambda i: (0, i, 0, 0)),
            pl.BlockSpec((1, D), lambda i: (0, 0)),
        ],
        out_specs=pl.BlockSpec((RB, D), lambda i: (i, 0)),
        out_shape=jax.ShapeDtypeStruct((NP, D), jnp.float32),
    )(accp, deg4, bias)


def kernel(x, edge_index, W1, b1, W2, b2, W3, b3):
    N, D = x.shape
    E = edge_index.shape[1]
    NP = -(-N // RB) * RB
    NT = NC * NS
    tot = -(-E // (NT * 16 * CH)) * 16 * NC
    c0pt = max(16, int(round(tot * C0_SHARE / 16.0)) * 16)
    c1pt = tot - c0pt
    chunks = (c0pt + c1pt) * NS
    ET = chunks * CH
    PAD = ET - E

    i32 = jnp.int32
    pad_iota = jnp.arange(PAD, dtype=i32)
    pad_rows = N + pad_iota % (NP - N)
    src = jnp.concatenate([edge_index[0].astype(i32), pad_rows])
    dst = jnp.concatenate([edge_index[1].astype(i32), pad_rows])
    src2 = src.reshape(chunks, CH)
    dst2 = dst.reshape(chunks, CH)
    xp = jnp.pad(x, ((0, NP - N), (0, 0)))
    z2 = jnp.zeros((NP, D), jnp.float32)
    z1 = jnp.zeros((NP,), jnp.float32)

    sc_deg = _make_sc_deg(chunks, NP)
    sc_mp = _make_sc_mp(c0pt, c1pt, NP, D)

    degp = sc_deg(dst2, z1)
    deg4 = degp.reshape(NC, NP // RB, 1, RB)

    m1 = _tc_first(xp, deg4, W1, NP, D)
    a1 = sc_mp(src2, dst2, m1, z2)
    m2 = _tc_mid(a1, deg4, W2, b1.reshape(1, D), NP, D)
    a2 = sc_mp(src2, dst2, m2, z2)
    m3 = _tc_mid(a2, deg4, W3, b2.reshape(1, D), NP, D)
    a3 = sc_mp(src2, dst2, m3, z2)
    out = _tc_last(a3, deg4, b3.reshape(1, D), NP, D)
    return out[:N][None, :, :]
